# baseline (speedup 1.0000x reference)
import functools

import jax
import jax.numpy as jnp
from jax import lax
from jax.experimental import pallas as pl
from jax.experimental.pallas import tpu as pltpu
from jax.experimental.pallas import tpu_sc as plsc

_N = 8192
_K = 50
_EMB = 32
_D = 3 * _EMB
_P = _N * _K
_NW = 32
_PPW = _P // _NW
_GRP = _PPW // 16
_R = 64


def _sc_indices(packed_hbm, speaker_hbm, bucket_hbm, idx_hbm,
                packed_v, speaker_v, bucket_v, idx_w):
    sid = lax.axis_index("s")
    w = sid * 2 + lax.axis_index("c")
    base_w = w * _PPW

    pltpu.sync_copy(packed_hbm.at[pl.ds(base_w, _PPW)], packed_v)
    pltpu.sync_copy(speaker_hbm, speaker_v)
    pltpu.sync_copy(bucket_hbm, bucket_v)

    def group(j, carry):
        pk = packed_v[pl.ds(j * 16, 16)]
        t = pk & 8191
        i = (pk >> 13) & 8191
        s_i = pk >> 26
        s_t = plsc.load_gather(speaker_v, [t])
        same = (s_i == s_t).astype(jnp.int32)
        d = jnp.maximum(i - t, 1)
        bucket = plsc.load_gather(bucket_v, [d])
        idx_w[pl.ds(j * 16, 16)] = same * 9 + bucket
        return carry

    lax.fori_loop(0, _GRP, group, 0)
    pltpu.sync_copy(idx_w, idx_hbm.at[pl.ds(base_w, _PPW)])


def _tc_expand(idx_ref, tbl_ref, out_ref):
    oh = (idx_ref[...][:, :, None] ==
          lax.broadcasted_iota(jnp.int32, (1, 1, 18), 2)).astype(jnp.float32)
    flat = oh.reshape(_R * _K, 18)
    res = jnp.dot(flat, tbl_ref[...], preferred_element_type=jnp.float32)
    out_ref[...] = res.reshape(_R, _K, _D)


@jax.jit
def kernel(top_indices, speaker_map, speaker_emb, distance_emb, genre_emb):
    table = jnp.concatenate(
        [
            jnp.repeat(speaker_emb, 9, axis=0),
            jnp.tile(distance_emb, (2, 1)),
            jnp.broadcast_to(genre_emb[0:1], (18, _EMB)),
        ],
        axis=1,
    )
    top_flat = top_indices.reshape(_P).astype(jnp.int32)
    wid_flat = jnp.repeat(jnp.arange(_N, dtype=jnp.int32), _K)
    spk_flat = jnp.repeat(speaker_map.astype(jnp.int32), _K)
    packed_flat = top_flat | (wid_flat << 13) | (spk_flat << 26)
    dd = jnp.maximum(jnp.arange(_N, dtype=jnp.int32), 1)
    bucket_lut = jnp.where(
        dd < 5, dd - 1,
        jnp.minimum(
            jnp.floor(jnp.log2(dd.astype(jnp.float32))), 6.0
        ).astype(jnp.int32) + 2)

    mesh = plsc.VectorSubcoreMesh(core_axis_name="c", subcore_axis_name="s")
    idx_flat = pl.kernel(
        _sc_indices,
        out_type=jax.ShapeDtypeStruct((_P,), jnp.int32),
        mesh=mesh,
        scratch_types=[
            pltpu.VMEM((_PPW,), jnp.int32),
            pltpu.VMEM((_N,), jnp.int32),
            pltpu.VMEM((_N,), jnp.int32),
            pltpu.VMEM((_PPW,), jnp.int32),
        ],
        compiler_params=pltpu.CompilerParams(
            use_tc_tiling_on_sc=False, needs_layout_passes=False),
    )(packed_flat, speaker_map.astype(jnp.int32), bucket_lut)

    out = pl.pallas_call(
        _tc_expand,
        grid=(_N // _R,),
        in_specs=[
            pl.BlockSpec((_R, _K), lambda g: (g, 0)),
            pl.BlockSpec((18, _D), lambda g: (0, 0)),
        ],
        out_specs=pl.BlockSpec((_R, _K, _D), lambda g: (g, 0, 0)),
        out_shape=jax.ShapeDtypeStruct((_N, _K, _D), jnp.float32),
    )(idx_flat.reshape(_N, _K), table)
    return out
